# baseline (speedup 1.0000x reference)
import functools

import jax
import jax.numpy as jnp
from jax import lax
from jax.experimental import pallas as pl
from jax.experimental.pallas import tpu as pltpu
from jax.experimental.pallas import tpu_sc as plsc

VOCAB = 100000
EMBED_DIM = 32
BATCH = 1024


_INFO = plsc.get_sparse_core_info()
_NC, _NS = _INFO.num_cores, _INFO.num_subcores
_NW = _NC * _NS
_B_PER_W = BATCH // _NW


def _make_sc_gather():
  mesh = plsc.VectorSubcoreMesh(core_axis_name="c", subcore_axis_name="s")

  @functools.partial(
      pl.kernel,
      mesh=mesh,
      out_type=jax.ShapeDtypeStruct((BATCH, EMBED_DIM), jnp.float32),
      scratch_types=[
          pltpu.VMEM((_B_PER_W,), jnp.int32),
          pltpu.VMEM((_B_PER_W, EMBED_DIM), jnp.float32),
          pltpu.SemaphoreType.DMA,
      ],
      compiler_params=pltpu.CompilerParams(use_tc_tiling_on_sc=False),
  )
  def gather(table_hbm, idx_hbm, out_hbm, idx_v, rows_v, sem):
    wid = lax.axis_index("s") * _NC + lax.axis_index("c")
    base = wid * _B_PER_W
    pltpu.sync_copy(idx_hbm.at[pl.ds(base, _B_PER_W)], idx_v)
    pltpu.async_copy(table_hbm.at[idx_v], rows_v, sem).wait()
    pltpu.sync_copy(rows_v, out_hbm.at[pl.ds(base, _B_PER_W)])

  return gather


_sc_gather = _make_sc_gather()


_BV = 2048


def _proj_kernel(e_ref, w_ref, b_ref, o_ref):
  o_ref[...] = (
      jnp.dot(e_ref[...], w_ref[...], preferred_element_type=jnp.float32)
      + b_ref[...]
  )


def _tc_proj(embeds, W, b2d):
  nv = pl.cdiv(VOCAB, _BV)
  return pl.pallas_call(
      _proj_kernel,
      grid=(nv,),
      in_specs=[
          pl.BlockSpec((BATCH, EMBED_DIM), lambda v: (0, 0)),
          pl.BlockSpec((EMBED_DIM, _BV), lambda v: (0, v)),
          pl.BlockSpec((1, _BV), lambda v: (0, v)),
      ],
      out_specs=pl.BlockSpec((BATCH, _BV), lambda v: (0, v)),
      out_shape=jax.ShapeDtypeStruct((BATCH, VOCAB), jnp.float32),
      compiler_params=pltpu.CompilerParams(
          dimension_semantics=("arbitrary",),
      ),
  )(embeds, W, b2d)


def kernel(inputs, emb_table, W, b):
  idx = inputs.astype(jnp.int32)
  embeds = _sc_gather(emb_table, idx)
  return _tc_proj(embeds, W, b.reshape(1, VOCAB))
